# baseline (speedup 1.0000x reference)
import dataclasses
import functools

import jax
import jax.numpy as jnp
from jax import lax
from jax.experimental import pallas as pl
from jax.experimental.pallas import tpu as pltpu
from jax.experimental.pallas import tpu_sc as plsc

N = 10000
D_IN = 128
D_HID = 16
N_CLASSES = 32
E = 320000

NC = 2
NS = 16
NW = NC * NS
LANES = 16

NPAD = 10240
C = 128
CPAIR = 160
CH0 = 102
CH1 = CPAIR - CH0
CH_MAX = max(CH0, CH1)
TOT_CHUNKS = NS * CPAIR
EPAD = TOT_CHUNKS * C
ROWS_PER_TILE = NPAD // NS

def _sc_compiler_params():
    return pltpu.CompilerParams(needs_layout_passes=False,
                                use_tc_tiling_on_sc=False)


@functools.cache
def _make_deg_sc():
    mesh = plsc.VectorSubcoreMesh(core_axis_name="c", subcore_axis_name="s")
    return functools.partial(
        pl.kernel,
        out_type=jax.ShapeDtypeStruct((NW, NPAD), jnp.float32),
        mesh=mesh,
        compiler_params=_sc_compiler_params(),
        scratch_types=[
            pltpu.VMEM((NPAD,), jnp.float32),
            pltpu.VMEM((CH_MAX, C), jnp.int32),
            pltpu.VMEM((CH_MAX, C), jnp.float32),
            pltpu.SemaphoreType.DMA,
        ],
    )(_deg_sc_body)


def _deg_sc_body(col_hbm, w_hbm, out_hbm, deg_v, col_v, w_v, sem):
    cid = lax.axis_index("c")
    sid = lax.axis_index("s")
    wid = sid * NC + cid
    base = sid * CPAIR + cid * CH0

    @pl.loop(0, NPAD, step=LANES)
    def _(i):
        deg_v[pl.ds(i, LANES)] = jnp.zeros((LANES,), jnp.float32)

    def run(nch):
        def go():
            cp_c = pltpu.make_async_copy(col_hbm.at[pl.ds(base, nch)],
                                         col_v.at[pl.ds(0, nch)], sem)
            cp_w = pltpu.make_async_copy(w_hbm.at[pl.ds(base, nch)],
                                         w_v.at[pl.ds(0, nch)], sem)
            cp_c.start()
            cp_w.start()
            cp_c.wait()
            cp_w.wait()

            @pl.loop(0, nch)
            def _(kk):
                @pl.loop(0, C, step=LANES)
                def _(e):
                    idx = col_v[kk, pl.ds(e, LANES)]
                    val = w_v[kk, pl.ds(e, LANES)]
                    plsc.addupdate_scatter(deg_v, [idx], val)
        return go

    pl.when(cid == 0)(run(CH0))
    pl.when(cid == 1)(run(CH1))

    pltpu.sync_copy(deg_v, out_hbm.at[wid])


@functools.cache
def _make_agg(D, bf16_gather=False):
    mesh = plsc.VectorSubcoreMesh(core_axis_name="c", subcore_axis_name="s")
    gdt = jnp.bfloat16 if bf16_gather else jnp.float32

    @functools.partial(
        pl.kernel,
        out_type=jax.ShapeDtypeStruct((NC, NPAD, D), jnp.float32),
        mesh=mesh,
        compiler_params=_sc_compiler_params(),
        scratch_types=[
            pltpu.VMEM((CH_MAX, C), jnp.int32),
            pltpu.VMEM((CH_MAX, C), jnp.int32),
            pltpu.VMEM((CH_MAX, C), jnp.float32),
            pltpu.VMEM((C, D), gdt),
            pltpu.VMEM((C, D), gdt),
            pltpu.VMEM((C, D), jnp.float32),
            pltpu.VMEM((C, D), jnp.float32),
            pltpu.VMEM_SHARED((NPAD, D), jnp.float32),
            pltpu.SemaphoreType.DMA,
            pltpu.SemaphoreType.DMA,
            pltpu.SemaphoreType.DMA,
            pltpu.SemaphoreType.DMA,
            pltpu.SemaphoreType.DMA,
        ],
    )
    def _agg(g_hbm, row_hbm, col_hbm, w_hbm, out_hbm,
             row_v, col_v, w_v, g0, g1, s0, s1, acc_sh,
             sem_in, gsem0, gsem1, ssem0, ssem1):
        cid = lax.axis_index("c")
        sid = lax.axis_index("s")
        base = sid * CPAIR + cid * CH0

        for buf in (s0, s1):
            @pl.loop(0, C)
            def _(i, buf=buf):
                for j in range(D // LANES):
                    buf[i, pl.ds(LANES * j, LANES)] = jnp.zeros((LANES,), jnp.float32)

        @pl.loop(0, ROWS_PER_TILE, step=2 * C)
        def _(r):
            r0 = sid * ROWS_PER_TILE + r
            pltpu.sync_copy(s0, acc_sh.at[pl.ds(r0, C)])
            pltpu.sync_copy(s1, acc_sh.at[pl.ds(r0 + C, C)])

        plsc.subcore_barrier()

        def gather_start(k, buf, sem):
            pltpu.async_copy(g_hbm.at[row_v.at[k]], buf, sem)

        def gather_wait(buf, sem):
            pltpu.make_async_copy(g_hbm.at[row_v.at[0]], buf, sem).wait()

        def scale(k, gbuf, sbuf):
            @pl.loop(0, C, step=LANES)
            def _(e):
                wv = w_v[k, pl.ds(e, LANES)]
                for l in range(LANES):
                    f = wv[l]
                    if bf16_gather:
                        for j in range(D // (2 * LANES)):
                            rowseg = gbuf[e + l, pl.ds(2 * LANES * j, 2 * LANES)]
                            a, b = plsc.unpack(
                                rowseg, format=plsc.PackFormat.INTERLEAVED)
                            half = D // 2
                            sbuf[e + l, pl.ds(LANES * j, LANES)] = a * f
                            sbuf[e + l, pl.ds(half + LANES * j, LANES)] = b * f
                    else:
                        for j in range(D // LANES):
                            sl = pl.ds(LANES * j, LANES)
                            sbuf[e + l, sl] = gbuf[e + l, sl] * f

        def scatter_start(k, sbuf, sem):
            pltpu.async_copy(sbuf, acc_sh.at[col_v.at[k]], sem, add=True)

        def scatter_wait(sbuf, sem):
            pltpu.make_async_copy(sbuf, acc_sh.at[col_v.at[0]], sem).wait()

        def run(nch):
            def go():
                cp_r = pltpu.make_async_copy(row_hbm.at[pl.ds(base, nch)],
                                             row_v.at[pl.ds(0, nch)], sem_in)
                cp_c = pltpu.make_async_copy(col_hbm.at[pl.ds(base, nch)],
                                             col_v.at[pl.ds(0, nch)], sem_in)
                cp_w = pltpu.make_async_copy(w_hbm.at[pl.ds(base, nch)],
                                             w_v.at[pl.ds(0, nch)], sem_in)
                cp_r.start()
                cp_c.start()
                cp_w.start()
                cp_r.wait()
                cp_c.wait()
                cp_w.wait()

                gather_start(0, g0, gsem0)
                gather_start(1, g1, gsem1)

                gather_wait(g0, gsem0)
                scale(0, g0, s0)
                gather_start(2, g0, gsem0)
                scatter_start(0, s0, ssem0)
                gather_wait(g1, gsem1)
                scale(1, g1, s1)
                gather_start(3, g1, gsem1)
                scatter_start(1, s1, ssem1)

                @pl.loop(0, (nch - 4) // 2)
                def _(t):
                    k = 2 * t + 2
                    scatter_wait(s0, ssem0)
                    gather_wait(g0, gsem0)
                    scale(k, g0, s0)
                    gather_start(k + 2, g0, gsem0)
                    scatter_start(k, s0, ssem0)
                    scatter_wait(s1, ssem1)
                    gather_wait(g1, gsem1)
                    scale(k + 1, g1, s1)
                    gather_start(k + 3, g1, gsem1)
                    scatter_start(k + 1, s1, ssem1)

                k_tail = nch - 2
                scatter_wait(s0, ssem0)
                gather_wait(g0, gsem0)
                scale(k_tail, g0, s0)
                scatter_start(k_tail, s0, ssem0)
                scatter_wait(s1, ssem1)
                gather_wait(g1, gsem1)
                scale(k_tail + 1, g1, s1)
                scatter_start(k_tail + 1, s1, ssem1)
                scatter_wait(s0, ssem0)
                scatter_wait(s1, ssem1)
            return go

        pl.when(cid == 0)(run(CH0))
        pl.when(cid == 1)(run(CH1))

        plsc.subcore_barrier()
        r0 = sid * ROWS_PER_TILE
        pltpu.sync_copy(acc_sh.at[pl.ds(r0, ROWS_PER_TILE)],
                        out_hbm.at[cid, pl.ds(r0, ROWS_PER_TILE)])

    return _agg


RB = 2560


def _tc_mm1_body(x_ref, w1_ref, h1_ref):
    h1_ref[...] = jnp.dot(x_ref[...], w1_ref[...],
                          preferred_element_type=jnp.float32)


def _tc_a_body(deg_ref, h1_ref, dis_ref, g1_ref):
    deg = jnp.sum(deg_ref[...], axis=0) + 1.0
    dis = jnp.where(deg > 0, lax.rsqrt(deg), 0.0)[:, None]
    dis_ref[...] = dis
    g1_ref[...] = h1_ref[...] * dis


def _tc_b_body(dis_ref, s1_ref, h1_ref, b1_ref, w2_ref, h2_ref, g2_ref):
    dis = dis_ref[...]
    s1 = s1_ref[0] + s1_ref[1]
    pre = dis * s1 + (dis * dis) * h1_ref[...] + b1_ref[...]
    a1 = jnp.maximum(pre, 0.0)
    h2 = jnp.dot(a1, w2_ref[...], preferred_element_type=jnp.float32)
    h2_ref[...] = h2
    g2 = h2 * dis
    half = N_CLASSES // 2
    g2i = jnp.stack([g2[:, :half], g2[:, half:]], axis=2).reshape(RB, N_CLASSES)
    g2_ref[...] = g2i.astype(jnp.bfloat16)


def _tc_c_body(dis_ref, s2_ref, h2_ref, b2_ref, out_ref):
    dis = dis_ref[...]
    s2 = s2_ref[0] + s2_ref[1]
    pre = dis * s2 + (dis * dis) * h2_ref[...] + b2_ref[...]
    m = jnp.max(pre, axis=1, keepdims=True)
    t = pre - m
    out_ref[...] = t - jnp.log(jnp.sum(jnp.exp(t), axis=1, keepdims=True))


def _row_spec(d):
    return pl.BlockSpec((RB, d), lambda i: (i, 0))


def _part_spec(d):
    return pl.BlockSpec((NC, RB, d), lambda i: (0, i, 0))


def _full_spec(shape):
    return pl.BlockSpec(shape, lambda i: tuple(0 for _ in shape))


@jax.jit
def kernel(x, edge_index, edge_weight, W1, b1, W2, b2):
    row = edge_index[0].astype(jnp.int32)
    col = edge_index[1].astype(jnp.int32)
    w = edge_weight.astype(jnp.float32)
    pad = EPAD - row.shape[0]
    row = jnp.concatenate([row, jnp.zeros((pad,), jnp.int32)])
    col = jnp.concatenate([col, jnp.zeros((pad,), jnp.int32)])
    w = jnp.concatenate([w, jnp.zeros((pad,), jnp.float32)])
    row = row.reshape(TOT_CHUNKS, C)
    col = col.reshape(TOT_CHUNKS, C)
    w = w.reshape(TOT_CHUNKS, C)

    grid = (NPAD // RB,)
    f32 = jnp.float32

    deg_parts = _make_deg_sc()(col, w)

    xp = jnp.concatenate([x, jnp.zeros((NPAD - N, D_IN), x.dtype)])
    h1 = pl.pallas_call(
        _tc_mm1_body,
        grid=grid,
        in_specs=[pl.BlockSpec((RB, D_IN), lambda i: (i, 0)),
                  _full_spec((D_IN, D_HID))],
        out_specs=_row_spec(D_HID),
        out_shape=jax.ShapeDtypeStruct((NPAD, D_HID), f32),
    )(xp, W1)

    dis, g1 = pl.pallas_call(
        _tc_a_body,
        grid=grid,
        in_specs=[pl.BlockSpec((NW, RB), lambda i: (0, i)), _row_spec(D_HID)],
        out_specs=[_row_spec(1), _row_spec(D_HID)],
        out_shape=(jax.ShapeDtypeStruct((NPAD, 1), f32),
                   jax.ShapeDtypeStruct((NPAD, D_HID), f32)),
    )(deg_parts, h1)

    s1 = _make_agg(D_HID)(g1, row, col, w)

    h2, g2 = pl.pallas_call(
        _tc_b_body,
        grid=grid,
        in_specs=[_row_spec(1), _part_spec(D_HID), _row_spec(D_HID),
                  _full_spec((1, D_HID)), _full_spec((D_HID, N_CLASSES))],
        out_specs=[_row_spec(N_CLASSES), _row_spec(N_CLASSES)],
        out_shape=(jax.ShapeDtypeStruct((NPAD, N_CLASSES), f32),
                   jax.ShapeDtypeStruct((NPAD, N_CLASSES), jnp.bfloat16)),
    )(dis, s1, h1, b1.reshape(1, D_HID), W2)

    s2 = _make_agg(N_CLASSES, True)(g2, row, col, w)

    out = pl.pallas_call(
        _tc_c_body,
        grid=grid,
        in_specs=[_row_spec(1), _part_spec(N_CLASSES), _row_spec(N_CLASSES),
                  _full_spec((1, N_CLASSES))],
        out_specs=_row_spec(N_CLASSES),
        out_shape=jax.ShapeDtypeStruct((NPAD, N_CLASSES), f32),
    )(dis, s2, h2, b2.reshape(1, N_CLASSES))
    return out[:N]


# speedup vs baseline: 1.1017x; 1.1017x over previous
import dataclasses
import functools

import jax
import jax.numpy as jnp
from jax import lax
from jax.experimental import pallas as pl
from jax.experimental.pallas import tpu as pltpu
from jax.experimental.pallas import tpu_sc as plsc

N = 10000
D_IN = 128
D_HID = 16
N_CLASSES = 32
E = 320000

NC = 2
NS = 16
NW = NC * NS
LANES = 16

NPAD = 10240
C = 128
CPAIR = 160
CH0 = 102
CH1 = CPAIR - CH0
CH_MAX = max(CH0, CH1)
TOT_CHUNKS = NS * CPAIR
EPAD = TOT_CHUNKS * C
ROWS_PER_TILE = NPAD // NS

def _sc_compiler_params():
    return pltpu.CompilerParams(needs_layout_passes=False,
                                use_tc_tiling_on_sc=False)


@functools.cache
def _make_deg_sc():
    mesh = plsc.VectorSubcoreMesh(core_axis_name="c", subcore_axis_name="s")
    return functools.partial(
        pl.kernel,
        out_type=jax.ShapeDtypeStruct((NW, NPAD), jnp.float32),
        mesh=mesh,
        compiler_params=_sc_compiler_params(),
        scratch_types=[
            pltpu.VMEM((NPAD,), jnp.float32),
            pltpu.VMEM((CH_MAX, C), jnp.int32),
            pltpu.VMEM((CH_MAX, C), jnp.float32),
            pltpu.SemaphoreType.DMA,
        ],
    )(_deg_sc_body)


def _deg_sc_body(col_hbm, w_hbm, out_hbm, deg_v, col_v, w_v, sem):
    cid = lax.axis_index("c")
    sid = lax.axis_index("s")
    wid = sid * NC + cid
    base = sid * CPAIR + cid * CH0

    @pl.loop(0, NPAD, step=LANES)
    def _(i):
        deg_v[pl.ds(i, LANES)] = jnp.zeros((LANES,), jnp.float32)

    def run(nch):
        def go():
            cp_c = pltpu.make_async_copy(col_hbm.at[pl.ds(base, nch)],
                                         col_v.at[pl.ds(0, nch)], sem)
            cp_w = pltpu.make_async_copy(w_hbm.at[pl.ds(base, nch)],
                                         w_v.at[pl.ds(0, nch)], sem)
            cp_c.start()
            cp_w.start()
            cp_c.wait()
            cp_w.wait()

            @pl.loop(0, nch)
            def _(kk):
                @pl.loop(0, C, step=LANES)
                def _(e):
                    idx = col_v[kk, pl.ds(e, LANES)]
                    val = w_v[kk, pl.ds(e, LANES)]
                    plsc.addupdate_scatter(deg_v, [idx], val)
        return go

    pl.when(cid == 0)(run(CH0))
    pl.when(cid == 1)(run(CH1))

    pltpu.sync_copy(deg_v, out_hbm.at[wid])


@functools.cache
def _make_agg(D):
    mesh = plsc.VectorSubcoreMesh(core_axis_name="c", subcore_axis_name="s")
    gdt = jnp.float32

    @functools.partial(
        pl.kernel,
        out_type=jax.ShapeDtypeStruct((NC, NPAD, D), jnp.float32),
        mesh=mesh,
        compiler_params=_sc_compiler_params(),
        scratch_types=[
            pltpu.VMEM((CH_MAX, C), jnp.int32),
            pltpu.VMEM((CH_MAX, C), jnp.int32),
            pltpu.VMEM((CH_MAX, C), jnp.float32),
            pltpu.VMEM((C, D), gdt),
            pltpu.VMEM((C, D), gdt),
            pltpu.VMEM((C, D), jnp.float32),
            pltpu.VMEM((C, D), jnp.float32),
            pltpu.VMEM_SHARED((NPAD, D), jnp.float32),
            pltpu.SemaphoreType.DMA,
            pltpu.SemaphoreType.DMA,
            pltpu.SemaphoreType.DMA,
            pltpu.SemaphoreType.DMA,
            pltpu.SemaphoreType.DMA,
        ],
    )
    def _agg(g_hbm, row_hbm, col_hbm, w_hbm, out_hbm,
             row_v, col_v, w_v, g0, g1, s0, s1, acc_sh,
             sem_in, gsem0, gsem1, ssem0, ssem1):
        cid = lax.axis_index("c")
        sid = lax.axis_index("s")
        base = sid * CPAIR + cid * CH0

        for buf in (s0, s1):
            @pl.loop(0, C)
            def _(i, buf=buf):
                for j in range(D // LANES):
                    buf[i, pl.ds(LANES * j, LANES)] = jnp.zeros((LANES,), jnp.float32)

        @pl.loop(0, ROWS_PER_TILE, step=2 * C)
        def _(r):
            r0 = sid * ROWS_PER_TILE + r
            pltpu.sync_copy(s0, acc_sh.at[pl.ds(r0, C)])
            pltpu.sync_copy(s1, acc_sh.at[pl.ds(r0 + C, C)])

        plsc.subcore_barrier()

        def gather_start(k, buf, sem):
            pltpu.async_copy(g_hbm.at[row_v.at[k]], buf, sem)

        def gather_wait(buf, sem):
            pltpu.make_async_copy(g_hbm.at[row_v.at[0]], buf, sem).wait()

        def scale(k, gbuf, sbuf):
            @plsc.parallel_loop(0, C, LANES, unroll=2)
            def _(e):
                wv = w_v[k, pl.ds(e, LANES)]
                for l in range(LANES):
                    f = wv[l]
                    for j in range(D // LANES):
                        sl = pl.ds(LANES * j, LANES)
                        sbuf[e + l, sl] = gbuf[e + l, sl] * f

        def scatter_start(k, sbuf, sem):
            pltpu.async_copy(sbuf, acc_sh.at[col_v.at[k]], sem, add=True)

        def scatter_wait(sbuf, sem):
            pltpu.make_async_copy(sbuf, acc_sh.at[col_v.at[0]], sem).wait()

        def run(nch):
            def go():
                cp_r = pltpu.make_async_copy(row_hbm.at[pl.ds(base, nch)],
                                             row_v.at[pl.ds(0, nch)], sem_in)
                cp_c = pltpu.make_async_copy(col_hbm.at[pl.ds(base, nch)],
                                             col_v.at[pl.ds(0, nch)], sem_in)
                cp_w = pltpu.make_async_copy(w_hbm.at[pl.ds(base, nch)],
                                             w_v.at[pl.ds(0, nch)], sem_in)
                cp_r.start()
                cp_c.start()
                cp_w.start()
                cp_r.wait()
                cp_c.wait()
                cp_w.wait()

                gather_start(0, g0, gsem0)
                gather_start(1, g1, gsem1)

                gather_wait(g0, gsem0)
                scale(0, g0, s0)
                gather_start(2, g0, gsem0)
                scatter_start(0, s0, ssem0)
                gather_wait(g1, gsem1)
                scale(1, g1, s1)
                gather_start(3, g1, gsem1)
                scatter_start(1, s1, ssem1)

                @pl.loop(0, (nch - 4) // 2)
                def _(t):
                    k = 2 * t + 2
                    scatter_wait(s0, ssem0)
                    gather_wait(g0, gsem0)
                    scale(k, g0, s0)
                    gather_start(k + 2, g0, gsem0)
                    scatter_start(k, s0, ssem0)
                    scatter_wait(s1, ssem1)
                    gather_wait(g1, gsem1)
                    scale(k + 1, g1, s1)
                    gather_start(k + 3, g1, gsem1)
                    scatter_start(k + 1, s1, ssem1)

                k_tail = nch - 2
                scatter_wait(s0, ssem0)
                gather_wait(g0, gsem0)
                scale(k_tail, g0, s0)
                scatter_start(k_tail, s0, ssem0)
                scatter_wait(s1, ssem1)
                gather_wait(g1, gsem1)
                scale(k_tail + 1, g1, s1)
                scatter_start(k_tail + 1, s1, ssem1)
                scatter_wait(s0, ssem0)
                scatter_wait(s1, ssem1)
            return go

        pl.when(cid == 0)(run(CH0))
        pl.when(cid == 1)(run(CH1))

        plsc.subcore_barrier()
        r0 = sid * ROWS_PER_TILE
        pltpu.sync_copy(acc_sh.at[pl.ds(r0, ROWS_PER_TILE)],
                        out_hbm.at[cid, pl.ds(r0, ROWS_PER_TILE)])

    return _agg


RB = 2560


def _tc_mm1_body(x_ref, w1_ref, h1_ref):
    h1_ref[...] = jnp.dot(x_ref[...], w1_ref[...],
                          preferred_element_type=jnp.float32)


def _tc_a_body(deg_ref, h1_ref, dis_ref, g1_ref):
    deg = jnp.sum(deg_ref[...], axis=0) + 1.0
    dis = jnp.where(deg > 0, lax.rsqrt(deg), 0.0)[:, None]
    dis_ref[...] = dis
    g1_ref[...] = h1_ref[...] * dis


def _tc_b_body(dis_ref, s1_ref, h1_ref, b1_ref, w2_ref, h2_ref, g2_ref):
    dis = dis_ref[...]
    s1 = s1_ref[0] + s1_ref[1]
    pre = dis * s1 + (dis * dis) * h1_ref[...] + b1_ref[...]
    a1 = jnp.maximum(pre, 0.0)
    h2 = jnp.dot(a1, w2_ref[...], preferred_element_type=jnp.float32)
    h2_ref[...] = h2
    g2_ref[...] = h2 * dis


def _tc_c_body(dis_ref, s2_ref, h2_ref, b2_ref, out_ref):
    dis = dis_ref[...]
    s2 = s2_ref[0] + s2_ref[1]
    pre = dis * s2 + (dis * dis) * h2_ref[...] + b2_ref[...]
    m = jnp.max(pre, axis=1, keepdims=True)
    t = pre - m
    out_ref[...] = t - jnp.log(jnp.sum(jnp.exp(t), axis=1, keepdims=True))


def _row_spec(d):
    return pl.BlockSpec((RB, d), lambda i: (i, 0))


def _part_spec(d):
    return pl.BlockSpec((NC, RB, d), lambda i: (0, i, 0))


def _full_spec(shape):
    return pl.BlockSpec(shape, lambda i: tuple(0 for _ in shape))


@jax.jit
def kernel(x, edge_index, edge_weight, W1, b1, W2, b2):
    row = edge_index[0].astype(jnp.int32)
    col = edge_index[1].astype(jnp.int32)
    w = edge_weight.astype(jnp.float32)
    pad = EPAD - row.shape[0]
    row = jnp.concatenate([row, jnp.zeros((pad,), jnp.int32)])
    col = jnp.concatenate([col, jnp.zeros((pad,), jnp.int32)])
    w = jnp.concatenate([w, jnp.zeros((pad,), jnp.float32)])
    row = row.reshape(TOT_CHUNKS, C)
    col = col.reshape(TOT_CHUNKS, C)
    w = w.reshape(TOT_CHUNKS, C)

    grid = (NPAD // RB,)
    f32 = jnp.float32

    deg_parts = _make_deg_sc()(col, w)

    xp = jnp.concatenate([x, jnp.zeros((NPAD - N, D_IN), x.dtype)])
    h1 = pl.pallas_call(
        _tc_mm1_body,
        grid=grid,
        in_specs=[pl.BlockSpec((RB, D_IN), lambda i: (i, 0)),
                  _full_spec((D_IN, D_HID))],
        out_specs=_row_spec(D_HID),
        out_shape=jax.ShapeDtypeStruct((NPAD, D_HID), f32),
    )(xp, W1)

    dis, g1 = pl.pallas_call(
        _tc_a_body,
        grid=grid,
        in_specs=[pl.BlockSpec((NW, RB), lambda i: (0, i)), _row_spec(D_HID)],
        out_specs=[_row_spec(1), _row_spec(D_HID)],
        out_shape=(jax.ShapeDtypeStruct((NPAD, 1), f32),
                   jax.ShapeDtypeStruct((NPAD, D_HID), f32)),
    )(deg_parts, h1)

    s1 = _make_agg(D_HID)(g1, row, col, w)

    h2, g2 = pl.pallas_call(
        _tc_b_body,
        grid=grid,
        in_specs=[_row_spec(1), _part_spec(D_HID), _row_spec(D_HID),
                  _full_spec((1, D_HID)), _full_spec((D_HID, N_CLASSES))],
        out_specs=[_row_spec(N_CLASSES), _row_spec(N_CLASSES)],
        out_shape=(jax.ShapeDtypeStruct((NPAD, N_CLASSES), f32),
                   jax.ShapeDtypeStruct((NPAD, N_CLASSES), f32)),
    )(dis, s1, h1, b1.reshape(1, D_HID), W2)

    s2 = _make_agg(N_CLASSES)(g2, row, col, w)

    out = pl.pallas_call(
        _tc_c_body,
        grid=grid,
        in_specs=[_row_spec(1), _part_spec(N_CLASSES), _row_spec(N_CLASSES),
                  _full_spec((1, N_CLASSES))],
        out_specs=_row_spec(N_CLASSES),
        out_shape=jax.ShapeDtypeStruct((NPAD, N_CLASSES), f32),
    )(dis, s2, h2, b2.reshape(1, N_CLASSES))
    return out[:N]


# speedup vs baseline: 1.1451x; 1.0394x over previous
import dataclasses
import functools

import jax
import jax.numpy as jnp
from jax import lax
from jax.experimental import pallas as pl
from jax.experimental.pallas import tpu as pltpu
from jax.experimental.pallas import tpu_sc as plsc

N = 10000
D_IN = 128
D_HID = 16
N_CLASSES = 32
E = 320000

NC = 2
NS = 16
NW = NC * NS
LANES = 16

NPAD = 10240
C = 128
CPAIR = 160
CH0_DEG = 110
CH0_A16 = 104
CH0_A32 = 108
TOT_CHUNKS = NS * CPAIR
EPAD = TOT_CHUNKS * C
ROWS_PER_TILE = NPAD // NS

def _sc_compiler_params():
    return pltpu.CompilerParams(needs_layout_passes=False,
                                use_tc_tiling_on_sc=False)


@functools.cache
def _make_deg_sc():
    mesh = plsc.VectorSubcoreMesh(core_axis_name="c", subcore_axis_name="s")
    chmax = max(CH0_DEG, CPAIR - CH0_DEG)
    return functools.partial(
        pl.kernel,
        out_type=jax.ShapeDtypeStruct((NW, NPAD), jnp.float32),
        mesh=mesh,
        compiler_params=_sc_compiler_params(),
        scratch_types=[
            pltpu.VMEM((NPAD,), jnp.float32),
            pltpu.VMEM((chmax, C), jnp.int32),
            pltpu.VMEM((chmax, C), jnp.float32),
            pltpu.SemaphoreType.DMA,
        ],
    )(_deg_sc_body)


def _deg_sc_body(col_hbm, w_hbm, out_hbm, deg_v, col_v, w_v, sem):
    cid = lax.axis_index("c")
    sid = lax.axis_index("s")
    wid = sid * NC + cid
    base = sid * CPAIR + cid * CH0_DEG

    @pl.loop(0, NPAD, step=LANES)
    def _(i):
        deg_v[pl.ds(i, LANES)] = jnp.zeros((LANES,), jnp.float32)

    def run(nch):
        def go():
            cp_c = pltpu.make_async_copy(col_hbm.at[pl.ds(base, nch)],
                                         col_v.at[pl.ds(0, nch)], sem)
            cp_w = pltpu.make_async_copy(w_hbm.at[pl.ds(base, nch)],
                                         w_v.at[pl.ds(0, nch)], sem)
            cp_c.start()
            cp_w.start()
            cp_c.wait()
            cp_w.wait()

            @pl.loop(0, nch)
            def _(kk):
                @pl.loop(0, C, step=LANES)
                def _(e):
                    idx = col_v[kk, pl.ds(e, LANES)]
                    val = w_v[kk, pl.ds(e, LANES)]
                    plsc.addupdate_scatter(deg_v, [idx], val)
        return go

    pl.when(cid == 0)(run(CH0_DEG))
    pl.when(cid == 1)(run(CPAIR - CH0_DEG))

    pltpu.sync_copy(deg_v, out_hbm.at[wid])


@functools.cache
def _make_agg(D, ch0):
    mesh = plsc.VectorSubcoreMesh(core_axis_name="c", subcore_axis_name="s")
    gdt = jnp.float32
    chmax = max(ch0, CPAIR - ch0)

    @functools.partial(
        pl.kernel,
        out_type=jax.ShapeDtypeStruct((NC, NPAD, D), jnp.float32),
        mesh=mesh,
        compiler_params=_sc_compiler_params(),
        scratch_types=[
            pltpu.VMEM((chmax, C), jnp.int32),
            pltpu.VMEM((chmax, C), jnp.int32),
            pltpu.VMEM((chmax, C), jnp.float32),
            pltpu.VMEM((C, D), gdt),
            pltpu.VMEM((C, D), gdt),
            pltpu.VMEM((C, D), jnp.float32),
            pltpu.VMEM((C, D), jnp.float32),
            pltpu.VMEM_SHARED((NPAD, D), jnp.float32),
            pltpu.SemaphoreType.DMA,
            pltpu.SemaphoreType.DMA,
            pltpu.SemaphoreType.DMA,
            pltpu.SemaphoreType.DMA,
            pltpu.SemaphoreType.DMA,
        ],
    )
    def _agg(g_hbm, row_hbm, col_hbm, w_hbm, out_hbm,
             row_v, col_v, w_v, g0, g1, s0, s1, acc_sh,
             sem_in, gsem0, gsem1, ssem0, ssem1):
        cid = lax.axis_index("c")
        sid = lax.axis_index("s")
        base = sid * CPAIR + cid * ch0

        for buf in (s0, s1):
            @pl.loop(0, C)
            def _(i, buf=buf):
                for j in range(D // LANES):
                    buf[i, pl.ds(LANES * j, LANES)] = jnp.zeros((LANES,), jnp.float32)

        @pl.loop(0, ROWS_PER_TILE, step=2 * C)
        def _(r):
            r0 = sid * ROWS_PER_TILE + r
            pltpu.sync_copy(s0, acc_sh.at[pl.ds(r0, C)])
            pltpu.sync_copy(s1, acc_sh.at[pl.ds(r0 + C, C)])

        plsc.subcore_barrier()

        def gather_start(k, buf, sem):
            pltpu.async_copy(g_hbm.at[row_v.at[k]], buf, sem)

        def gather_wait(buf, sem):
            pltpu.make_async_copy(g_hbm.at[row_v.at[0]], buf, sem).wait()

        def scale(k, gbuf, sbuf):
            @pl.loop(0, C, step=LANES)
            def _(e):
                wv = w_v[k, pl.ds(e, LANES)]
                for l in range(LANES):
                    f = wv[l]
                    for j in range(D // LANES):
                        sl = pl.ds(LANES * j, LANES)
                        sbuf[e + l, sl] = gbuf[e + l, sl] * f

        def scatter_start(k, sbuf, sem):
            pltpu.async_copy(sbuf, acc_sh.at[col_v.at[k]], sem, add=True)

        def scatter_wait(sbuf, sem):
            pltpu.make_async_copy(sbuf, acc_sh.at[col_v.at[0]], sem).wait()

        def run(nch):
            def go():
                cp_r = pltpu.make_async_copy(row_hbm.at[pl.ds(base, nch)],
                                             row_v.at[pl.ds(0, nch)], sem_in)
                cp_c = pltpu.make_async_copy(col_hbm.at[pl.ds(base, nch)],
                                             col_v.at[pl.ds(0, nch)], sem_in)
                cp_w = pltpu.make_async_copy(w_hbm.at[pl.ds(base, nch)],
                                             w_v.at[pl.ds(0, nch)], sem_in)
                cp_r.start()
                cp_c.start()
                cp_w.start()
                cp_r.wait()
                cp_c.wait()
                cp_w.wait()

                gather_start(0, g0, gsem0)
                gather_start(1, g1, gsem1)

                gather_wait(g0, gsem0)
                scale(0, g0, s0)
                gather_start(2, g0, gsem0)
                scatter_start(0, s0, ssem0)
                gather_wait(g1, gsem1)
                scale(1, g1, s1)
                gather_start(3, g1, gsem1)
                scatter_start(1, s1, ssem1)

                @pl.loop(0, (nch - 4) // 2)
                def _(t):
                    k = 2 * t + 2
                    scatter_wait(s0, ssem0)
                    gather_wait(g0, gsem0)
                    scale(k, g0, s0)
                    gather_start(k + 2, g0, gsem0)
                    scatter_start(k, s0, ssem0)
                    scatter_wait(s1, ssem1)
                    gather_wait(g1, gsem1)
                    scale(k + 1, g1, s1)
                    gather_start(k + 3, g1, gsem1)
                    scatter_start(k + 1, s1, ssem1)

                k_tail = nch - 2
                scatter_wait(s0, ssem0)
                gather_wait(g0, gsem0)
                scale(k_tail, g0, s0)
                scatter_start(k_tail, s0, ssem0)
                scatter_wait(s1, ssem1)
                gather_wait(g1, gsem1)
                scale(k_tail + 1, g1, s1)
                scatter_start(k_tail + 1, s1, ssem1)
                scatter_wait(s0, ssem0)
                scatter_wait(s1, ssem1)
            return go

        pl.when(cid == 0)(run(ch0))
        pl.when(cid == 1)(run(CPAIR - ch0))

        plsc.subcore_barrier()
        r0 = sid * ROWS_PER_TILE
        pltpu.sync_copy(acc_sh.at[pl.ds(r0, ROWS_PER_TILE)],
                        out_hbm.at[cid, pl.ds(r0, ROWS_PER_TILE)])

    return _agg


RB = 2560


def _tc_mm1_body(x_ref, w1_ref, h1_ref):
    h1_ref[...] = jnp.dot(x_ref[...], w1_ref[...],
                          preferred_element_type=jnp.float32)


def _tc_a_body(deg_ref, h1_ref, dis_ref, g1_ref):
    deg = jnp.sum(deg_ref[...], axis=0) + 1.0
    dis = jnp.where(deg > 0, lax.rsqrt(deg), 0.0)[:, None]
    dis_ref[...] = dis
    g1_ref[...] = h1_ref[...] * dis


def _tc_b_body(dis_ref, s1_ref, h1_ref, b1_ref, w2_ref, h2_ref, g2_ref):
    dis = dis_ref[...]
    s1 = s1_ref[0] + s1_ref[1]
    pre = dis * s1 + (dis * dis) * h1_ref[...] + b1_ref[...]
    a1 = jnp.maximum(pre, 0.0)
    h2 = jnp.dot(a1, w2_ref[...], preferred_element_type=jnp.float32)
    h2_ref[...] = h2
    g2_ref[...] = h2 * dis


def _tc_c_body(dis_ref, s2_ref, h2_ref, b2_ref, out_ref):
    dis = dis_ref[...]
    s2 = s2_ref[0] + s2_ref[1]
    pre = dis * s2 + (dis * dis) * h2_ref[...] + b2_ref[...]
    m = jnp.max(pre, axis=1, keepdims=True)
    t = pre - m
    out_ref[...] = t - jnp.log(jnp.sum(jnp.exp(t), axis=1, keepdims=True))


def _row_spec(d):
    return pl.BlockSpec((RB, d), lambda i: (i, 0))


def _part_spec(d):
    return pl.BlockSpec((NC, RB, d), lambda i: (0, i, 0))


def _full_spec(shape):
    return pl.BlockSpec(shape, lambda i: tuple(0 for _ in shape))


@jax.jit
def kernel(x, edge_index, edge_weight, W1, b1, W2, b2):
    row = edge_index[0].astype(jnp.int32)
    col = edge_index[1].astype(jnp.int32)
    w = edge_weight.astype(jnp.float32)
    pad = EPAD - row.shape[0]
    row = jnp.concatenate([row, jnp.zeros((pad,), jnp.int32)])
    col = jnp.concatenate([col, jnp.zeros((pad,), jnp.int32)])
    w = jnp.concatenate([w, jnp.zeros((pad,), jnp.float32)])
    row = row.reshape(TOT_CHUNKS, C)
    col = col.reshape(TOT_CHUNKS, C)
    w = w.reshape(TOT_CHUNKS, C)

    grid = (NPAD // RB,)
    f32 = jnp.float32

    deg_parts = _make_deg_sc()(col, w)

    xp = jnp.concatenate([x, jnp.zeros((NPAD - N, D_IN), x.dtype)])
    h1 = pl.pallas_call(
        _tc_mm1_body,
        grid=grid,
        in_specs=[pl.BlockSpec((RB, D_IN), lambda i: (i, 0)),
                  _full_spec((D_IN, D_HID))],
        out_specs=_row_spec(D_HID),
        out_shape=jax.ShapeDtypeStruct((NPAD, D_HID), f32),
    )(xp, W1)

    dis, g1 = pl.pallas_call(
        _tc_a_body,
        grid=grid,
        in_specs=[pl.BlockSpec((NW, RB), lambda i: (0, i)), _row_spec(D_HID)],
        out_specs=[_row_spec(1), _row_spec(D_HID)],
        out_shape=(jax.ShapeDtypeStruct((NPAD, 1), f32),
                   jax.ShapeDtypeStruct((NPAD, D_HID), f32)),
    )(deg_parts, h1)

    s1 = _make_agg(D_HID, CH0_A16)(g1, row, col, w)

    h2, g2 = pl.pallas_call(
        _tc_b_body,
        grid=grid,
        in_specs=[_row_spec(1), _part_spec(D_HID), _row_spec(D_HID),
                  _full_spec((1, D_HID)), _full_spec((D_HID, N_CLASSES))],
        out_specs=[_row_spec(N_CLASSES), _row_spec(N_CLASSES)],
        out_shape=(jax.ShapeDtypeStruct((NPAD, N_CLASSES), f32),
                   jax.ShapeDtypeStruct((NPAD, N_CLASSES), f32)),
    )(dis, s1, h1, b1.reshape(1, D_HID), W2)

    s2 = _make_agg(N_CLASSES, CH0_A32)(g2, row, col, w)

    out = pl.pallas_call(
        _tc_c_body,
        grid=grid,
        in_specs=[_row_spec(1), _part_spec(N_CLASSES), _row_spec(N_CLASSES),
                  _full_spec((1, N_CLASSES))],
        out_specs=_row_spec(N_CLASSES),
        out_shape=jax.ShapeDtypeStruct((NPAD, N_CLASSES), f32),
    )(dis, s2, h2, b2.reshape(1, N_CLASSES))
    return out[:N]
